# initial kernel scaffold (unmeasured)
import functools

import jax
import jax.numpy as jnp
from jax import lax
from jax.experimental import pallas as pl
from jax.experimental.pallas import tpu as pltpu

N_DEV = 4
SQ = 2048
SKV_PER = 2048
H_PER = 8
DH = 128
DM = 1024
KBLK = 512
SCALE = 0.08838834764831843


def _attn_body(x_ref, wq_ref, wo_ref, k_hbm, v_hbm, out_ref,
               kc, vc, send_sems, recv_sems, copy_sems):
    me = lax.axis_index("i")

    bsem = pltpu.get_barrier_semaphore()
    for p in range(N_DEV - 1):
        j = (me + 1 + p) % N_DEV
        pl.semaphore_signal(bsem, 1, device_id=(j,),
                            device_id_type=pl.DeviceIdType.MESH)
    pl.semaphore_wait(bsem, N_DEV - 1)

    sends = []
    for p in range(N_DEV - 1):
        j = (me + 1 + p) % N_DEV
        for t, (src, chunks) in enumerate(((k_hbm, kc), (v_hbm, vc))):
            rdma = pltpu.make_async_remote_copy(
                src_ref=src.at[0, :, pl.ds(H_PER * j, H_PER), :],
                dst_ref=chunks.at[me],
                send_sem=send_sems.at[p, t],
                recv_sem=recv_sems.at[me, t],
                device_id=(j,),
                device_id_type=pl.DeviceIdType.MESH,
            )
            rdma.start()
            sends.append(rdma)

    local_cps = []
    for t, (src, chunks) in enumerate(((k_hbm, kc), (v_hbm, vc))):
        cp = pltpu.make_async_copy(
            src.at[0, :, pl.ds(H_PER * me, H_PER), :], chunks.at[me],
            copy_sems.at[t])
        cp.start()
        local_cps.append(cp)

    x2 = x_ref[0]
    q_all = jnp.dot(x2, wq_ref[...],
                    preferred_element_type=jnp.float32) * SCALE

    for cp in local_cps:
        cp.wait()
    for p in range(N_DEV - 1):
        j = (me + 1 + p) % N_DEV
        for t, (src, chunks) in enumerate(((k_hbm, kc), (v_hbm, vc))):
            recv = pltpu.make_async_remote_copy(
                src_ref=src.at[0, :, pl.ds(0, H_PER), :],
                dst_ref=chunks.at[j],
                send_sem=send_sems.at[p, t],
                recv_sem=recv_sems.at[j, t],
                device_id=(j,),
                device_id_type=pl.DeviceIdType.MESH,
            )
            recv.wait_recv()

    n_blocks = (N_DEV * SKV_PER) // KBLK
    per_chunk = SKV_PER // KBLK
    qi = lax.broadcasted_iota(jnp.int32, (SQ, KBLK), 0)
    col = lax.broadcasted_iota(jnp.int32, (SQ, KBLK), 1)

    out_acc = jnp.zeros((SQ, DM), jnp.float32)
    for h in range(H_PER):
        q_h = q_all[:, h * DH:(h + 1) * DH]

        def kb_body(kb, carry, q_h=q_h, h=h):
            m, l, acc = carry
            slot = kb // per_chunk
            sb = kb % per_chunk
            kblk = kc[slot, pl.ds(sb * KBLK, KBLK), h, :]
            vblk = vc[slot, pl.ds(sb * KBLK, KBLK), h, :]
            s = jnp.dot(q_h, kblk.T, preferred_element_type=jnp.float32)
            ki = slot * SKV_PER + sb * KBLK + col
            mask = (jnp.abs(qi - ki) <= 128) | (ki < 32) | (qi < 32)
            s = jnp.where(mask, s, -1e9)
            m_new = jnp.maximum(m, s.max(axis=1, keepdims=True))
            pw = jnp.exp(s - m_new)
            corr = jnp.exp(m - m_new)
            l_new = l * corr + pw.sum(axis=1, keepdims=True)
            acc_new = acc * corr + jnp.dot(
                pw, vblk, preferred_element_type=jnp.float32)
            return m_new, l_new, acc_new

        m0 = jnp.full((SQ, 1), -1e30, jnp.float32)
        l0 = jnp.zeros((SQ, 1), jnp.float32)
        a0 = jnp.zeros((SQ, DH), jnp.float32)
        m, l, acc = lax.fori_loop(0, n_blocks, kb_body, (m0, l0, a0))
        ctx_h = acc / l
        out_acc = out_acc + jnp.dot(
            ctx_h, wo_ref[h * DH:(h + 1) * DH, :],
            preferred_element_type=jnp.float32)

    out_ref[...] = out_acc

    for rdma in sends:
        rdma.wait_send()


def _allreduce_body(part_ref, out_ref, rbuf, ssems, rsems):
    me = lax.axis_index("i")

    bsem = pltpu.get_barrier_semaphore()
    for p in range(N_DEV - 1):
        j = (me + 1 + p) % N_DEV
        pl.semaphore_signal(bsem, 1, device_id=(j,),
                            device_id_type=pl.DeviceIdType.MESH)
    pl.semaphore_wait(bsem, N_DEV - 1)

    sends = []
    for p in range(N_DEV - 1):
        j = (me + 1 + p) % N_DEV
        slot_on_j = (me - j - 1) % N_DEV
        rdma = pltpu.make_async_remote_copy(
            src_ref=part_ref,
            dst_ref=rbuf.at[slot_on_j],
            send_sem=ssems.at[p],
            recv_sem=rsems.at[slot_on_j],
            device_id=(j,),
            device_id_type=pl.DeviceIdType.MESH,
        )
        rdma.start()
        sends.append(rdma)

    total = part_ref[...]
    for d in range(N_DEV - 1):
        recv = pltpu.make_async_remote_copy(
            src_ref=part_ref,
            dst_ref=rbuf.at[d],
            send_sem=ssems.at[0],
            recv_sem=rsems.at[d],
            device_id=(me,),
            device_id_type=pl.DeviceIdType.MESH,
        )
        recv.wait_recv()
        total = total + rbuf[d]

    out_ref[0] = total

    for rdma in sends:
        rdma.wait_send()


def kernel(x, Wq, K_ext, V_ext, Wo):
    part = pl.pallas_call(
        _attn_body,
        out_shape=jax.ShapeDtypeStruct((SQ, DM), jnp.float32),
        in_specs=[
            pl.BlockSpec(memory_space=pltpu.VMEM),
            pl.BlockSpec(memory_space=pltpu.VMEM),
            pl.BlockSpec(memory_space=pltpu.VMEM),
            pl.BlockSpec(memory_space=pl.ANY),
            pl.BlockSpec(memory_space=pl.ANY),
        ],
        out_specs=pl.BlockSpec(memory_space=pltpu.VMEM),
        scratch_shapes=[
            pltpu.VMEM((N_DEV, SKV_PER, H_PER, DH), jnp.float32),
            pltpu.VMEM((N_DEV, SKV_PER, H_PER, DH), jnp.float32),
            pltpu.SemaphoreType.DMA((N_DEV - 1, 2)),
            pltpu.SemaphoreType.DMA((N_DEV, 2)),
            pltpu.SemaphoreType.DMA((2,)),
        ],
        compiler_params=pltpu.CompilerParams(collective_id=0),
    )(x, Wq, Wo, K_ext, V_ext)

    out = pl.pallas_call(
        _allreduce_body,
        out_shape=jax.ShapeDtypeStruct((1, SQ, DM), jnp.float32),
        in_specs=[pl.BlockSpec(memory_space=pltpu.VMEM)],
        out_specs=pl.BlockSpec(memory_space=pltpu.VMEM),
        scratch_shapes=[
            pltpu.VMEM((N_DEV - 1, SQ, DM), jnp.float32),
            pltpu.SemaphoreType.DMA((N_DEV - 1,)),
            pltpu.SemaphoreType.DMA((N_DEV - 1,)),
        ],
        compiler_params=pltpu.CompilerParams(collective_id=1),
    )(part)
    return out


# baseline (device time: 1056614 ns/iter reference)
import jax
import jax.numpy as jnp
from jax import lax
from jax.experimental import pallas as pl
from jax.experimental.pallas import tpu as pltpu

N_DEV = 4
SQ = 2048
SKV_PER = 2048
SKV = N_DEV * SKV_PER
H_PER = 8
DH = 128
DM = 1024
KBLK = 512
QBLK = 512
N_KB = SKV // KBLK
KB_PER_CHUNK = SKV_PER // KBLK
SCALE = 0.08838834764831843


def _attn_body(x_ref, wq_ref, wo_ref, k_hbm, v_hbm, out_ref, kc, vc,
               kbuf, vbuf, send_sems, recv_sems, copy_sems, stream_sems):
    me = lax.axis_index("i")

    bsem = pltpu.get_barrier_semaphore()
    for p in range(N_DEV - 1):
        j = (me + 1 + p) % N_DEV
        pl.semaphore_signal(bsem, 1, device_id=(j,),
                            device_id_type=pl.DeviceIdType.MESH)
    pl.semaphore_wait(bsem, N_DEV - 1)

    sends = []
    for p in range(N_DEV - 1):
        j = (me + 1 + p) % N_DEV
        for t, (src, chunks) in enumerate(((k_hbm, kc), (v_hbm, vc))):
            rdma = pltpu.make_async_remote_copy(
                src_ref=src.at[0, :, pl.ds(H_PER * j, H_PER), :],
                dst_ref=chunks.at[me],
                send_sem=send_sems.at[p, t],
                recv_sem=recv_sems.at[me, t],
                device_id=(j,),
                device_id_type=pl.DeviceIdType.MESH,
            )
            rdma.start()
            sends.append(rdma)

    local_cps = []
    for t, (src, chunks) in enumerate(((k_hbm, kc), (v_hbm, vc))):
        cp = pltpu.make_async_copy(
            src.at[0, :, pl.ds(H_PER * me, H_PER), :], chunks.at[me],
            copy_sems.at[t])
        cp.start()
        local_cps.append(cp)

    q_all = jnp.dot(x_ref[0], wq_ref[...],
                    preferred_element_type=jnp.float32) * SCALE

    for cp in local_cps:
        cp.wait()
    for p in range(N_DEV - 1):
        j = (me + 1 + p) % N_DEV
        for t, (src, chunks) in enumerate(((k_hbm, kc), (v_hbm, vc))):
            recv = pltpu.make_async_remote_copy(
                src_ref=src.at[0, :, pl.ds(0, H_PER), :],
                dst_ref=chunks.at[j],
                send_sem=send_sems.at[p, t],
                recv_sem=recv_sems.at[j, t],
                device_id=(j,),
                device_id_type=pl.DeviceIdType.MESH,
            )
            recv.wait_recv()

    def stream_desc(h, kb, b):
        chunk = kb // KB_PER_CHUNK
        row = (kb % KB_PER_CHUNK) * KBLK
        return [
            pltpu.make_async_copy(
                chunks.at[chunk, pl.ds(row, KBLK), h, :],
                buf.at[b], stream_sems.at[b, t])
            for t, (chunks, buf) in enumerate(((kc, kbuf), (vc, vbuf)))
        ]

    for cp in stream_desc(0, 0, 0):
        cp.start()

    for h in range(H_PER):
        q_h = q_all[:, h * DH:(h + 1) * DH]

        def kb_body(kb, carry, h=h, q_h=q_h):
            m, l, acc = carry
            b = lax.rem(kb, 2)
            for cp in stream_desc(h, kb, b):
                cp.wait()

            @pl.when(kb + 1 < N_KB)
            def _():
                for cp in stream_desc(h, kb + 1, 1 - b):
                    cp.start()

            kblk = kbuf[b]
            vblk = vbuf[b]
            s = jnp.dot(q_h, kblk.T, preferred_element_type=jnp.float32)
            qi = lax.broadcasted_iota(jnp.int32, (SQ, KBLK), 0)
            ki = kb * KBLK + lax.broadcasted_iota(jnp.int32, (SQ, KBLK), 1)
            mask = (jnp.abs(qi - ki) <= 128) | (ki < 32) | (qi < 32)
            s = jnp.where(mask, s, -1e9)
            m_new = jnp.maximum(m, s.max(axis=1, keepdims=True))
            pw = jnp.exp(s - m_new)
            corr = jnp.exp(m - m_new)
            l_new = l * corr + pw.sum(axis=1, keepdims=True)
            acc_new = acc * corr + jnp.dot(
                pw, vblk, preferred_element_type=jnp.float32)
            return m_new, l_new, acc_new

        m0 = jnp.full((SQ, 1), -1e30, jnp.float32)
        l0 = jnp.zeros((SQ, 1), jnp.float32)
        a0 = jnp.zeros((SQ, DH), jnp.float32)
        m, l, acc = lax.fori_loop(0, N_KB, kb_body, (m0, l0, a0))

        if h + 1 < H_PER:
            for cp in stream_desc(h + 1, 0, 0):
                cp.start()

        ctx_h = acc / l
        for rb in range(SQ // QBLK):
            rows = pl.ds(rb * QBLK, QBLK)
            contrib = jnp.dot(ctx_h[rb * QBLK:(rb + 1) * QBLK, :],
                              wo_ref[h * DH:(h + 1) * DH, :],
                              preferred_element_type=jnp.float32)
            if h == 0:
                out_ref[rows, :] = contrib
            else:
                out_ref[rows, :] = out_ref[rows, :] + contrib

    for rdma in sends:
        rdma.wait_send()


def _allreduce_body(part_ref, out_ref, rbuf, ssems, rsems):
    me = lax.axis_index("i")

    bsem = pltpu.get_barrier_semaphore()
    for p in range(N_DEV - 1):
        j = (me + 1 + p) % N_DEV
        pl.semaphore_signal(bsem, 1, device_id=(j,),
                            device_id_type=pl.DeviceIdType.MESH)
    pl.semaphore_wait(bsem, N_DEV - 1)

    sends = []
    for p in range(N_DEV - 1):
        j = (me + 1 + p) % N_DEV
        slot_on_j = (me - j - 1) % N_DEV
        rdma = pltpu.make_async_remote_copy(
            src_ref=part_ref,
            dst_ref=rbuf.at[slot_on_j],
            send_sem=ssems.at[p],
            recv_sem=rsems.at[slot_on_j],
            device_id=(j,),
            device_id_type=pl.DeviceIdType.MESH,
        )
        rdma.start()
        sends.append(rdma)

    for d in range(N_DEV - 1):
        recv = pltpu.make_async_remote_copy(
            src_ref=part_ref,
            dst_ref=rbuf.at[d],
            send_sem=ssems.at[0],
            recv_sem=rsems.at[d],
            device_id=(me,),
            device_id_type=pl.DeviceIdType.MESH,
        )
        recv.wait_recv()

    for rb in range(SQ // QBLK):
        rows = pl.ds(rb * QBLK, QBLK)
        out_ref[0, rows, :] = (
            (part_ref[rows, :] + rbuf[0, rows, :])
            + (rbuf[1, rows, :] + rbuf[2, rows, :])
        )

    for rdma in sends:
        rdma.wait_send()


def kernel(x, Wq, K_ext, V_ext, Wo):
    chunk_shape = jax.ShapeDtypeStruct((N_DEV, SKV_PER, H_PER, DH),
                                       jnp.float32)
    part, _, _ = pl.pallas_call(
        _attn_body,
        out_shape=[
            jax.ShapeDtypeStruct((SQ, DM), jnp.float32),
            chunk_shape,
            chunk_shape,
        ],
        in_specs=[
            pl.BlockSpec(memory_space=pltpu.VMEM),
            pl.BlockSpec(memory_space=pltpu.VMEM),
            pl.BlockSpec(memory_space=pltpu.VMEM),
            pl.BlockSpec(memory_space=pl.ANY),
            pl.BlockSpec(memory_space=pl.ANY),
        ],
        out_specs=[
            pl.BlockSpec(memory_space=pltpu.VMEM),
            pl.BlockSpec(memory_space=pl.ANY),
            pl.BlockSpec(memory_space=pl.ANY),
        ],
        scratch_shapes=[
            pltpu.VMEM((2, KBLK, DH), jnp.float32),
            pltpu.VMEM((2, KBLK, DH), jnp.float32),
            pltpu.SemaphoreType.DMA((N_DEV - 1, 2)),
            pltpu.SemaphoreType.DMA((N_DEV, 2)),
            pltpu.SemaphoreType.DMA((2,)),
            pltpu.SemaphoreType.DMA((2, 2)),
        ],
        compiler_params=pltpu.CompilerParams(collective_id=0),
    )(x, Wq, Wo, K_ext, V_ext)

    out = pl.pallas_call(
        _allreduce_body,
        out_shape=jax.ShapeDtypeStruct((1, SQ, DM), jnp.float32),
        in_specs=[pl.BlockSpec(memory_space=pltpu.VMEM)],
        out_specs=pl.BlockSpec(memory_space=pltpu.VMEM),
        scratch_shapes=[
            pltpu.VMEM((N_DEV - 1, SQ, DM), jnp.float32),
            pltpu.SemaphoreType.DMA((N_DEV - 1,)),
            pltpu.SemaphoreType.DMA((N_DEV - 1,)),
        ],
        compiler_params=pltpu.CompilerParams(collective_id=1),
    )(part)
    return out


# device time: 931899 ns/iter; 1.1338x vs baseline; 1.1338x over previous
import jax
import jax.numpy as jnp
from jax import lax
from jax.experimental import pallas as pl
from jax.experimental.pallas import tpu as pltpu

N_DEV = 4
SQ = 2048
SKV_PER = 2048
SKV = N_DEV * SKV_PER
H_PER = 8
DH = 128
DM = 1024
KBLK = 512
QBLK = 512
N_KB = SKV // KBLK
KB_PER_CHUNK = SKV_PER // KBLK
SCALE = 0.08838834764831843


def _attn_body(x_ref, wq_ref, wo_ref, k_hbm, v_hbm, out_ref, kc, vc,
               kbuf, vbuf, send_sems, recv_sems, copy_sems, stream_sems):
    me = lax.axis_index("i")

    bsem = pltpu.get_barrier_semaphore()
    for p in range(N_DEV - 1):
        j = (me + 1 + p) % N_DEV
        pl.semaphore_signal(bsem, 1, device_id=(j,),
                            device_id_type=pl.DeviceIdType.MESH)
    pl.semaphore_wait(bsem, N_DEV - 1)

    sends = []
    for p in range(N_DEV - 1):
        j = (me + 1 + p) % N_DEV
        for t, (src, chunks) in enumerate(((k_hbm, kc), (v_hbm, vc))):
            rdma = pltpu.make_async_remote_copy(
                src_ref=src.at[0, :, pl.ds(H_PER * j, H_PER), :],
                dst_ref=chunks.at[me],
                send_sem=send_sems.at[p, t],
                recv_sem=recv_sems.at[me, t],
                device_id=(j,),
                device_id_type=pl.DeviceIdType.MESH,
            )
            rdma.start()
            sends.append(rdma)

    local_cps = []
    for t, (src, chunks) in enumerate(((k_hbm, kc), (v_hbm, vc))):
        cp = pltpu.make_async_copy(
            src.at[0, :, pl.ds(H_PER * me, H_PER), :], chunks.at[me],
            copy_sems.at[t])
        cp.start()
        local_cps.append(cp)

    q_all = jnp.dot(x_ref[0], wq_ref[...],
                    preferred_element_type=jnp.float32) * SCALE

    for cp in local_cps:
        cp.wait()
    for p in range(N_DEV - 1):
        j = (me + 1 + p) % N_DEV
        for t, (src, chunks) in enumerate(((k_hbm, kc), (v_hbm, vc))):
            recv = pltpu.make_async_remote_copy(
                src_ref=src.at[0, :, pl.ds(0, H_PER), :],
                dst_ref=chunks.at[j],
                send_sem=send_sems.at[p, t],
                recv_sem=recv_sems.at[j, t],
                device_id=(j,),
                device_id_type=pl.DeviceIdType.MESH,
            )
            recv.wait_recv()

    def stream_desc(h, kb, b):
        chunk = kb // KB_PER_CHUNK
        row = (kb % KB_PER_CHUNK) * KBLK
        return [
            pltpu.make_async_copy(
                chunks.at[chunk, pl.ds(row, KBLK), h, :],
                buf.at[b], stream_sems.at[b, t])
            for t, (chunks, buf) in enumerate(((kc, kbuf), (vc, vbuf)))
        ]

    for cp in stream_desc(0, 0, 0):
        cp.start()

    N_KB_A = (SQ + 512) // KBLK
    G = 32

    for h in range(H_PER):
        q_h = q_all[:, h * DH:(h + 1) * DH]

        def kb_a_body(kb, carry, h=h, q_h=q_h):
            m, l, acc = carry
            b = lax.rem(kb, 2)
            for cp in stream_desc(h, kb, b):
                cp.wait()

            @pl.when(kb + 1 < N_KB_A)
            def _():
                for cp in stream_desc(h, kb + 1, 1 - b):
                    cp.start()

            kblk = kbuf[b]
            vblk = vbuf[b]
            s = jnp.dot(q_h, kblk.T, preferred_element_type=jnp.float32)
            qi = lax.broadcasted_iota(jnp.int32, (SQ, KBLK), 0)
            ki = kb * KBLK + lax.broadcasted_iota(jnp.int32, (SQ, KBLK), 1)
            mask = (jnp.abs(qi - ki) <= 128) | (ki < G) | (qi < G)
            s = jnp.where(mask, s, -1e9)
            m_new = jnp.maximum(m, s.max(axis=1, keepdims=True))
            pw = jnp.exp(s - m_new)
            corr = jnp.exp(m - m_new)
            l_new = l * corr + pw.sum(axis=1, keepdims=True)
            acc_new = acc * corr + jnp.dot(
                pw, vblk, preferred_element_type=jnp.float32)
            return m_new, l_new, acc_new

        m0 = jnp.full((SQ, 1), -1e30, jnp.float32)
        l0 = jnp.zeros((SQ, 1), jnp.float32)
        a0 = jnp.zeros((SQ, DH), jnp.float32)
        m, l, acc = lax.fori_loop(0, N_KB_A, kb_a_body, (m0, l0, a0))

        for cp in stream_desc(h, N_KB_A, N_KB_A % 2):
            cp.start()
        q_g = q_h[0:G, :]

        def kb_b_body(kb, carry, h=h, q_g=q_g):
            m, l, acc = carry
            b = lax.rem(kb, 2)
            for cp in stream_desc(h, kb, b):
                cp.wait()

            @pl.when(kb + 1 < N_KB)
            def _():
                for cp in stream_desc(h, kb + 1, 1 - b):
                    cp.start()

            kblk = kbuf[b]
            vblk = vbuf[b]
            s = jnp.dot(q_g, kblk.T, preferred_element_type=jnp.float32)
            m_new = jnp.maximum(m, s.max(axis=1, keepdims=True))
            pw = jnp.exp(s - m_new)
            corr = jnp.exp(m - m_new)
            l_new = l * corr + pw.sum(axis=1, keepdims=True)
            acc_new = acc * corr + jnp.dot(
                pw, vblk, preferred_element_type=jnp.float32)
            return m_new, l_new, acc_new

        mg0 = jnp.full((G, 1), -1e30, jnp.float32)
        lg0 = jnp.zeros((G, 1), jnp.float32)
        ag0 = jnp.zeros((G, DH), jnp.float32)
        mg, lg, ag = lax.fori_loop(N_KB_A, N_KB, kb_b_body, (mg0, lg0, ag0))

        if h + 1 < H_PER:
            for cp in stream_desc(h + 1, 0, N_KB % 2):
                cp.start()

        m_a = m[0:G, :]
        mc = jnp.maximum(m_a, mg)
        wa = jnp.exp(m_a - mc)
        wb = jnp.exp(mg - mc)
        ctx_g = (acc[0:G, :] * wa + ag * wb) / (l[0:G, :] * wa + lg * wb)

        ctx_h = jnp.concatenate([ctx_g, acc[G:, :] / l[G:, :]], axis=0)
        for rb in range(SQ // QBLK):
            rows = pl.ds(rb * QBLK, QBLK)
            contrib = jnp.dot(ctx_h[rb * QBLK:(rb + 1) * QBLK, :],
                              wo_ref[h * DH:(h + 1) * DH, :],
                              preferred_element_type=jnp.float32)
            if h == 0:
                out_ref[rows, :] = contrib
            else:
                out_ref[rows, :] = out_ref[rows, :] + contrib

    for rdma in sends:
        rdma.wait_send()


def _allreduce_body(part_ref, out_ref, rbuf, ssems, rsems):
    me = lax.axis_index("i")

    bsem = pltpu.get_barrier_semaphore()
    for p in range(N_DEV - 1):
        j = (me + 1 + p) % N_DEV
        pl.semaphore_signal(bsem, 1, device_id=(j,),
                            device_id_type=pl.DeviceIdType.MESH)
    pl.semaphore_wait(bsem, N_DEV - 1)

    sends = []
    for p in range(N_DEV - 1):
        j = (me + 1 + p) % N_DEV
        slot_on_j = (me - j - 1) % N_DEV
        rdma = pltpu.make_async_remote_copy(
            src_ref=part_ref,
            dst_ref=rbuf.at[slot_on_j],
            send_sem=ssems.at[p],
            recv_sem=rsems.at[slot_on_j],
            device_id=(j,),
            device_id_type=pl.DeviceIdType.MESH,
        )
        rdma.start()
        sends.append(rdma)

    for d in range(N_DEV - 1):
        recv = pltpu.make_async_remote_copy(
            src_ref=part_ref,
            dst_ref=rbuf.at[d],
            send_sem=ssems.at[0],
            recv_sem=rsems.at[d],
            device_id=(me,),
            device_id_type=pl.DeviceIdType.MESH,
        )
        recv.wait_recv()

    for rb in range(SQ // QBLK):
        rows = pl.ds(rb * QBLK, QBLK)
        out_ref[0, rows, :] = (
            (part_ref[rows, :] + rbuf[0, rows, :])
            + (rbuf[1, rows, :] + rbuf[2, rows, :])
        )

    for rdma in sends:
        rdma.wait_send()


def kernel(x, Wq, K_ext, V_ext, Wo):
    chunk_shape = jax.ShapeDtypeStruct((N_DEV, SKV_PER, H_PER, DH),
                                       jnp.float32)
    part, _, _ = pl.pallas_call(
        _attn_body,
        out_shape=[
            jax.ShapeDtypeStruct((SQ, DM), jnp.float32),
            chunk_shape,
            chunk_shape,
        ],
        in_specs=[
            pl.BlockSpec(memory_space=pltpu.VMEM),
            pl.BlockSpec(memory_space=pltpu.VMEM),
            pl.BlockSpec(memory_space=pltpu.VMEM),
            pl.BlockSpec(memory_space=pl.ANY),
            pl.BlockSpec(memory_space=pl.ANY),
        ],
        out_specs=[
            pl.BlockSpec(memory_space=pltpu.VMEM),
            pl.BlockSpec(memory_space=pl.ANY),
            pl.BlockSpec(memory_space=pl.ANY),
        ],
        scratch_shapes=[
            pltpu.VMEM((2, KBLK, DH), jnp.float32),
            pltpu.VMEM((2, KBLK, DH), jnp.float32),
            pltpu.SemaphoreType.DMA((N_DEV - 1, 2)),
            pltpu.SemaphoreType.DMA((N_DEV, 2)),
            pltpu.SemaphoreType.DMA((2,)),
            pltpu.SemaphoreType.DMA((2, 2)),
        ],
        compiler_params=pltpu.CompilerParams(collective_id=0),
    )(x, Wq, Wo, K_ext, V_ext)

    out = pl.pallas_call(
        _allreduce_body,
        out_shape=jax.ShapeDtypeStruct((1, SQ, DM), jnp.float32),
        in_specs=[pl.BlockSpec(memory_space=pltpu.VMEM)],
        out_specs=pl.BlockSpec(memory_space=pltpu.VMEM),
        scratch_shapes=[
            pltpu.VMEM((N_DEV - 1, SQ, DM), jnp.float32),
            pltpu.SemaphoreType.DMA((N_DEV - 1,)),
            pltpu.SemaphoreType.DMA((N_DEV - 1,)),
        ],
        compiler_params=pltpu.CompilerParams(collective_id=1),
    )(part)
    return out


# device time: 709695 ns/iter; 1.4888x vs baseline; 1.3131x over previous
import jax
import jax.numpy as jnp
from jax import lax
from jax.experimental import pallas as pl
from jax.experimental.pallas import tpu as pltpu

N_DEV = 4
SQ = 2048
SKV_PER = 2048
SKV = N_DEV * SKV_PER
H_PER = 8
DH = 128
DM = 1024
KBLK = 512
QBLK = 512
N_KB = SKV // KBLK
KB_PER_CHUNK = SKV_PER // KBLK
N_KB_A = (SQ + 512) // KBLK
G = 32
SCALE = 0.08838834764831843


def _attn_body(x_ref, wq_ref, wo_ref, k_hbm, v_hbm, out_ref, kc, vc,
               kbuf, vbuf, send_sems, recv_sems, copy_sems, stream_sems):
    me = lax.axis_index("i")

    bsem = pltpu.get_barrier_semaphore()
    for p in range(N_DEV - 1):
        j = (me + 1 + p) % N_DEV
        pl.semaphore_signal(bsem, 1, device_id=(j,),
                            device_id_type=pl.DeviceIdType.MESH)
    pl.semaphore_wait(bsem, N_DEV - 1)

    sends = []
    for p in range(N_DEV - 1):
        j = (me + 1 + p) % N_DEV
        for t, (src, chunks) in enumerate(((k_hbm, kc), (v_hbm, vc))):
            rdma = pltpu.make_async_remote_copy(
                src_ref=src.at[pl.ds(H_PER * j, H_PER), :, :],
                dst_ref=chunks.at[:, me, :, :],
                send_sem=send_sems.at[p, t],
                recv_sem=recv_sems.at[me, t],
                device_id=(j,),
                device_id_type=pl.DeviceIdType.MESH,
            )
            rdma.start()
            sends.append(rdma)

    local_cps = []
    for t, (src, chunks) in enumerate(((k_hbm, kc), (v_hbm, vc))):
        cp = pltpu.make_async_copy(
            src.at[pl.ds(H_PER * me, H_PER), :, :], chunks.at[:, me, :, :],
            copy_sems.at[t])
        cp.start()
        local_cps.append(cp)

    q16 = (jnp.dot(x_ref[0], wq_ref[...],
                   preferred_element_type=jnp.float32)
           * SCALE).astype(jnp.bfloat16)

    def wait_chunk(c):
        for t, (src, chunks) in enumerate(((k_hbm, kc), (v_hbm, vc))):
            @pl.when(me == c)
            def _(t=t):
                local_cps[t].wait()

            @pl.when(me != c)
            def _(t=t, src=src, chunks=chunks):
                recv = pltpu.make_async_remote_copy(
                    src_ref=src.at[pl.ds(0, H_PER), :, :],
                    dst_ref=chunks.at[:, c, :, :],
                    send_sem=send_sems.at[0, t],
                    recv_sem=recv_sems.at[c, t],
                    device_id=(me,),
                    device_id_type=pl.DeviceIdType.MESH,
                )
                recv.wait_recv()

    def stream_desc(h, kb, b):
        chunk = kb // KB_PER_CHUNK
        row = (kb % KB_PER_CHUNK) * KBLK
        return [
            pltpu.make_async_copy(
                chunks.at[h, chunk, pl.ds(row, KBLK), :],
                buf.at[b], stream_sems.at[b, t])
            for t, (chunks, buf) in enumerate(((kc, kbuf), (vc, vbuf)))
        ]

    wait_chunk(0)
    wait_chunk(1)
    for cp in stream_desc(0, 0, 0):
        cp.start()

    strips = []
    for h in range(H_PER):
        q_h = q16[:, h * DH:(h + 1) * DH]

        def kb_a_body(kb, carry, h=h, q_h=q_h):
            m, l, acc = carry
            b = lax.rem(h * N_KB_A + kb, 2)
            for cp in stream_desc(h, kb, b):
                cp.wait()

            @pl.when(kb + 1 < N_KB_A)
            def _():
                for cp in stream_desc(h, kb + 1, 1 - b):
                    cp.start()

            kblk = kbuf[b]
            vblk = vbuf[b]
            s = jnp.dot(q_h, kblk.T, preferred_element_type=jnp.float32)
            qi = lax.broadcasted_iota(jnp.int32, (SQ, KBLK), 0)
            ki = kb * KBLK + lax.broadcasted_iota(jnp.int32, (SQ, KBLK), 1)
            mask = (jnp.abs(qi - ki) <= 128) | (ki < G) | (qi < G)
            s = jnp.where(mask, s, -1e9)
            m_new = jnp.maximum(m, s.max(axis=1, keepdims=True))
            pw = jnp.exp(s - m_new)
            corr = jnp.exp(m - m_new)
            l_new = l * corr + pw.sum(axis=1, keepdims=True)
            acc_new = acc * corr + jnp.dot(
                pw.astype(jnp.bfloat16), vblk,
                preferred_element_type=jnp.float32)
            return m_new, l_new, acc_new

        m0 = jnp.full((SQ, 1), -1e30, jnp.float32)
        l0 = jnp.zeros((SQ, 1), jnp.float32)
        a0 = jnp.zeros((SQ, DH), jnp.float32)
        m, l, acc = lax.fori_loop(0, N_KB_A, kb_a_body, (m0, l0, a0))

        if h + 1 < H_PER:
            for cp in stream_desc(h + 1, 0, ((h + 1) * N_KB_A) % 2):
                cp.start()

        strips.append((m[0:G, :], l[0:G, :], acc[0:G, :]))
        ctx_h = (acc / l).astype(jnp.bfloat16)
        for rb in range(SQ // QBLK):
            rows = pl.ds(rb * QBLK, QBLK)
            contrib = jnp.dot(ctx_h[rb * QBLK:(rb + 1) * QBLK, :],
                              wo_ref[h * DH:(h + 1) * DH, :],
                              preferred_element_type=jnp.float32)
            if h == 0:
                out_ref[rows, :] = contrib
            else:
                out_ref[rows, :] = out_ref[rows, :] + contrib

    wait_chunk(2)
    wait_chunk(3)
    for cp in stream_desc(0, N_KB_A, 0):
        cp.start()

    n_b = N_KB - N_KB_A
    fix = jnp.zeros((G, DM), jnp.float32)
    for h in range(H_PER):
        q_g = q16[0:G, h * DH:(h + 1) * DH]

        def kb_b_body(kb, carry, h=h, q_g=q_g):
            m, l, acc = carry
            b = lax.rem(h * n_b + kb - N_KB_A, 2)
            for cp in stream_desc(h, kb, b):
                cp.wait()

            @pl.when(kb + 1 < N_KB)
            def _():
                for cp in stream_desc(h, kb + 1, 1 - b):
                    cp.start()

            kblk = kbuf[b]
            vblk = vbuf[b]
            s = jnp.dot(q_g, kblk.T, preferred_element_type=jnp.float32)
            m_new = jnp.maximum(m, s.max(axis=1, keepdims=True))
            pw = jnp.exp(s - m_new)
            corr = jnp.exp(m - m_new)
            l_new = l * corr + pw.sum(axis=1, keepdims=True)
            acc_new = acc * corr + jnp.dot(
                pw.astype(jnp.bfloat16), vblk,
                preferred_element_type=jnp.float32)
            return m_new, l_new, acc_new

        mg0 = jnp.full((G, 1), -1e30, jnp.float32)
        lg0 = jnp.zeros((G, 1), jnp.float32)
        ag0 = jnp.zeros((G, DH), jnp.float32)
        mg, lg, ag = lax.fori_loop(N_KB_A, N_KB, kb_b_body, (mg0, lg0, ag0))

        if h + 1 < H_PER:
            for cp in stream_desc(h + 1, N_KB_A, ((h + 1) * n_b) % 2):
                cp.start()

        m_a, l_a, a_a = strips[h]
        mc = jnp.maximum(m_a, mg)
        wa = jnp.exp(m_a - mc)
        wb = jnp.exp(mg - mc)
        ctx_g = ((a_a * wa + ag * wb) / (l_a * wa + lg * wb)
                 ).astype(jnp.bfloat16)
        fix = fix + jnp.dot(ctx_g, wo_ref[h * DH:(h + 1) * DH, :],
                            preferred_element_type=jnp.float32)

    out_ref[0:G, :] = fix

    for rdma in sends:
        rdma.wait_send()


def _allreduce_body(part_ref, out_ref, rbuf, ssems, rsems):
    me = lax.axis_index("i")

    bsem = pltpu.get_barrier_semaphore()
    for p in range(N_DEV - 1):
        j = (me + 1 + p) % N_DEV
        pl.semaphore_signal(bsem, 1, device_id=(j,),
                            device_id_type=pl.DeviceIdType.MESH)
    pl.semaphore_wait(bsem, N_DEV - 1)

    sends = []
    for p in range(N_DEV - 1):
        j = (me + 1 + p) % N_DEV
        slot_on_j = (me - j - 1) % N_DEV
        rdma = pltpu.make_async_remote_copy(
            src_ref=part_ref,
            dst_ref=rbuf.at[slot_on_j],
            send_sem=ssems.at[p],
            recv_sem=rsems.at[slot_on_j],
            device_id=(j,),
            device_id_type=pl.DeviceIdType.MESH,
        )
        rdma.start()
        sends.append(rdma)

    for d in range(N_DEV - 1):
        recv = pltpu.make_async_remote_copy(
            src_ref=part_ref,
            dst_ref=rbuf.at[d],
            send_sem=ssems.at[0],
            recv_sem=rsems.at[d],
            device_id=(me,),
            device_id_type=pl.DeviceIdType.MESH,
        )
        recv.wait_recv()

    for rb in range(SQ // QBLK):
        rows = pl.ds(rb * QBLK, QBLK)
        out_ref[0, rows, :] = (
            (part_ref[rows, :] + rbuf[0, rows, :])
            + (rbuf[1, rows, :] + rbuf[2, rows, :])
        )

    for rdma in sends:
        rdma.wait_send()


def kernel(x, Wq, K_ext, V_ext, Wo):
    x16 = x.astype(jnp.bfloat16)
    wq16 = Wq.astype(jnp.bfloat16)
    wo16 = Wo.astype(jnp.bfloat16)
    k16 = jnp.transpose(K_ext[0], (1, 0, 2)).astype(jnp.bfloat16)
    v16 = jnp.transpose(V_ext[0], (1, 0, 2)).astype(jnp.bfloat16)

    chunk_shape = jax.ShapeDtypeStruct((H_PER, N_DEV, SKV_PER, DH),
                                       jnp.bfloat16)
    part, _, _ = pl.pallas_call(
        _attn_body,
        out_shape=[
            jax.ShapeDtypeStruct((SQ, DM), jnp.float32),
            chunk_shape,
            chunk_shape,
        ],
        in_specs=[
            pl.BlockSpec(memory_space=pltpu.VMEM),
            pl.BlockSpec(memory_space=pltpu.VMEM),
            pl.BlockSpec(memory_space=pltpu.VMEM),
            pl.BlockSpec(memory_space=pl.ANY),
            pl.BlockSpec(memory_space=pl.ANY),
        ],
        out_specs=[
            pl.BlockSpec(memory_space=pltpu.VMEM),
            pl.BlockSpec(memory_space=pl.ANY),
            pl.BlockSpec(memory_space=pl.ANY),
        ],
        scratch_shapes=[
            pltpu.VMEM((2, KBLK, DH), jnp.bfloat16),
            pltpu.VMEM((2, KBLK, DH), jnp.bfloat16),
            pltpu.SemaphoreType.DMA((N_DEV - 1, 2)),
            pltpu.SemaphoreType.DMA((N_DEV, 2)),
            pltpu.SemaphoreType.DMA((2,)),
            pltpu.SemaphoreType.DMA((2, 2)),
        ],
        compiler_params=pltpu.CompilerParams(collective_id=0),
    )(x16, wq16, wo16, k16, v16)

    out = pl.pallas_call(
        _allreduce_body,
        out_shape=jax.ShapeDtypeStruct((1, SQ, DM), jnp.float32),
        in_specs=[pl.BlockSpec(memory_space=pltpu.VMEM)],
        out_specs=pl.BlockSpec(memory_space=pltpu.VMEM),
        scratch_shapes=[
            pltpu.VMEM((N_DEV - 1, SQ, DM), jnp.float32),
            pltpu.SemaphoreType.DMA((N_DEV - 1,)),
            pltpu.SemaphoreType.DMA((N_DEV - 1,)),
        ],
        compiler_params=pltpu.CompilerParams(collective_id=1),
    )(part)
    return out


# device time: 620672 ns/iter; 1.7024x vs baseline; 1.1434x over previous
import jax
import jax.numpy as jnp
from jax import lax
from jax.experimental import pallas as pl
from jax.experimental.pallas import tpu as pltpu

N_DEV = 4
SQ = 2048
SKV_PER = 2048
SKV = N_DEV * SKV_PER
H_PER = 8
DH = 128
DM = 1024
KBLK = 512
QBLK = 512
N_KB = SKV // KBLK
KB_PER_CHUNK = SKV_PER // KBLK
N_KB_A = (SQ + 512) // KBLK
G = 32
SCALE = 0.08838834764831843


def _attn_body(x_ref, wq_ref, wo_ref, k_hbm, v_hbm, out_ref, kc, vc,
               kbuf, vbuf, send_sems, recv_sems, copy_sems, stream_sems):
    me = lax.axis_index("i")

    bsem = pltpu.get_barrier_semaphore()
    for p in range(N_DEV - 1):
        j = (me + 1 + p) % N_DEV
        pl.semaphore_signal(bsem, 1, device_id=(j,),
                            device_id_type=pl.DeviceIdType.MESH)
    pl.semaphore_wait(bsem, N_DEV - 1)

    sends = []
    for p in range(N_DEV - 1):
        j = (me + 1 + p) % N_DEV
        for t, (src, chunks) in enumerate(((k_hbm, kc), (v_hbm, vc))):
            rdma = pltpu.make_async_remote_copy(
                src_ref=src.at[pl.ds(H_PER * j, H_PER), :, :],
                dst_ref=chunks.at[:, me, :, :],
                send_sem=send_sems.at[p, t],
                recv_sem=recv_sems.at[me, t],
                device_id=(j,),
                device_id_type=pl.DeviceIdType.MESH,
            )
            rdma.start()
            sends.append(rdma)

    local_cps = []
    for t, (src, chunks) in enumerate(((k_hbm, kc), (v_hbm, vc))):
        cp = pltpu.make_async_copy(
            src.at[pl.ds(H_PER * me, H_PER), :, :], chunks.at[:, me, :, :],
            copy_sems.at[t])
        cp.start()
        local_cps.append(cp)

    q16 = (jnp.dot(x_ref[0], wq_ref[...],
                   preferred_element_type=jnp.float32)
           * SCALE).astype(jnp.bfloat16)

    def wait_chunk(c):
        for t, (src, chunks) in enumerate(((k_hbm, kc), (v_hbm, vc))):
            @pl.when(me == c)
            def _(t=t):
                local_cps[t].wait()

            @pl.when(me != c)
            def _(t=t, src=src, chunks=chunks):
                recv = pltpu.make_async_remote_copy(
                    src_ref=src.at[pl.ds(0, H_PER), :, :],
                    dst_ref=chunks.at[:, c, :, :],
                    send_sem=send_sems.at[0, t],
                    recv_sem=recv_sems.at[c, t],
                    device_id=(me,),
                    device_id_type=pl.DeviceIdType.MESH,
                )
                recv.wait_recv()

    def stream_desc(h, kb, b):
        chunk = kb // KB_PER_CHUNK
        row = (kb % KB_PER_CHUNK) * KBLK
        return [
            pltpu.make_async_copy(
                chunks.at[h, chunk, pl.ds(row, KBLK), :],
                buf.at[b], stream_sems.at[b, t])
            for t, (chunks, buf) in enumerate(((kc, kbuf), (vc, vbuf)))
        ]

    wait_chunk(0)
    wait_chunk(1)
    for cp in stream_desc(0, 0, 0):
        cp.start()

    strips = []
    for h in range(H_PER):
        q_h = q16[:, h * DH:(h + 1) * DH]

        def kb_a_body(kb, carry, h=h, q_h=q_h):
            m, l, acc = carry
            b = lax.rem(h * N_KB_A + kb, 2)
            for cp in stream_desc(h, kb, b):
                cp.wait()

            @pl.when(kb + 1 < N_KB_A)
            def _():
                for cp in stream_desc(h, kb + 1, 1 - b):
                    cp.start()

            kblk = kbuf[b]
            vblk = vbuf[b]
            s = jnp.dot(q_h, kblk.T, preferred_element_type=jnp.float32)
            qi = lax.broadcasted_iota(jnp.int32, (SQ, KBLK), 0)
            ki = kb * KBLK + lax.broadcasted_iota(jnp.int32, (SQ, KBLK), 1)
            mask = (jnp.abs(qi - ki) <= 128) | (ki < G) | (qi < G)
            s = jnp.where(mask, s, -1e9)
            m_new = jnp.maximum(m, s.max(axis=1, keepdims=True))
            pw = jnp.exp(s - m_new)
            corr = jnp.exp(m - m_new)
            l_new = l * corr + pw.sum(axis=1, keepdims=True)
            acc_new = acc * corr + jnp.dot(
                pw.astype(jnp.bfloat16), vblk,
                preferred_element_type=jnp.float32)
            return m_new, l_new, acc_new

        m0 = jnp.full((SQ, 1), -1e30, jnp.float32)
        l0 = jnp.zeros((SQ, 1), jnp.float32)
        a0 = jnp.zeros((SQ, DH), jnp.float32)
        m, l, acc = lax.fori_loop(0, N_KB_A, kb_a_body, (m0, l0, a0))

        if h + 1 < H_PER:
            for cp in stream_desc(h + 1, 0, ((h + 1) * N_KB_A) % 2):
                cp.start()

        strips.append((m[0:G, :], l[0:G, :], acc[0:G, :]))
        ctx_h = (acc / l).astype(jnp.bfloat16)
        for rb in range(SQ // QBLK):
            rows = pl.ds(rb * QBLK, QBLK)
            contrib = jnp.dot(ctx_h[rb * QBLK:(rb + 1) * QBLK, :],
                              wo_ref[h * DH:(h + 1) * DH, :],
                              preferred_element_type=jnp.float32)
            if h == 0:
                out_ref[rows, :] = contrib
            else:
                out_ref[rows, :] = out_ref[rows, :] + contrib

    wait_chunk(2)
    wait_chunk(3)
    for cp in stream_desc(0, N_KB_A, 0):
        cp.start()

    n_b = N_KB - N_KB_A
    fix = jnp.zeros((G, DM), jnp.float32)
    for h in range(H_PER):
        q_g = q16[0:G, h * DH:(h + 1) * DH]

        def kb_b_body(kb, carry, h=h, q_g=q_g):
            m, l, acc = carry
            b = lax.rem(h * n_b + kb - N_KB_A, 2)
            for cp in stream_desc(h, kb, b):
                cp.wait()

            @pl.when(kb + 1 < N_KB)
            def _():
                for cp in stream_desc(h, kb + 1, 1 - b):
                    cp.start()

            kblk = kbuf[b]
            vblk = vbuf[b]
            s = jnp.dot(q_g, kblk.T, preferred_element_type=jnp.float32)
            m_new = jnp.maximum(m, s.max(axis=1, keepdims=True))
            pw = jnp.exp(s - m_new)
            corr = jnp.exp(m - m_new)
            l_new = l * corr + pw.sum(axis=1, keepdims=True)
            acc_new = acc * corr + jnp.dot(
                pw.astype(jnp.bfloat16), vblk,
                preferred_element_type=jnp.float32)
            return m_new, l_new, acc_new

        mg0 = jnp.full((G, 1), -1e30, jnp.float32)
        lg0 = jnp.zeros((G, 1), jnp.float32)
        ag0 = jnp.zeros((G, DH), jnp.float32)
        mg, lg, ag = lax.fori_loop(N_KB_A, N_KB, kb_b_body, (mg0, lg0, ag0))

        if h + 1 < H_PER:
            for cp in stream_desc(h + 1, N_KB_A, ((h + 1) * n_b) % 2):
                cp.start()

        m_a, l_a, a_a = strips[h]
        mc = jnp.maximum(m_a, mg)
        wa = jnp.exp(m_a - mc)
        wb = jnp.exp(mg - mc)
        ctx_g = ((a_a * wa + ag * wb) / (l_a * wa + lg * wb)
                 ).astype(jnp.bfloat16)
        fix = fix + jnp.dot(ctx_g, wo_ref[h * DH:(h + 1) * DH, :],
                            preferred_element_type=jnp.float32)

    out_ref[0:G, :] = fix

    for rdma in sends:
        rdma.wait_send()


def _allreduce_body(part_ref, out_ref, rbuf, ssems, rsems):
    me = lax.axis_index("i")

    bsem = pltpu.get_barrier_semaphore()
    for p in range(N_DEV - 1):
        j = (me + 1 + p) % N_DEV
        pl.semaphore_signal(bsem, 1, device_id=(j,),
                            device_id_type=pl.DeviceIdType.MESH)
    pl.semaphore_wait(bsem, N_DEV - 1)

    sends = []
    for p in range(N_DEV - 1):
        j = (me + 1 + p) % N_DEV
        slot_on_j = (me - j - 1) % N_DEV
        rdma = pltpu.make_async_remote_copy(
            src_ref=part_ref,
            dst_ref=rbuf.at[slot_on_j],
            send_sem=ssems.at[p],
            recv_sem=rsems.at[slot_on_j],
            device_id=(j,),
            device_id_type=pl.DeviceIdType.MESH,
        )
        rdma.start()
        sends.append(rdma)

    for d in range(N_DEV - 1):
        recv = pltpu.make_async_remote_copy(
            src_ref=part_ref,
            dst_ref=rbuf.at[d],
            send_sem=ssems.at[0],
            recv_sem=rsems.at[d],
            device_id=(me,),
            device_id_type=pl.DeviceIdType.MESH,
        )
        recv.wait_recv()

    f32 = jnp.float32
    for rb in range(SQ // QBLK):
        rows = pl.ds(rb * QBLK, QBLK)
        out_ref[0, rows, :] = (
            (part_ref[rows, :].astype(f32) + rbuf[0, rows, :].astype(f32))
            + (rbuf[1, rows, :].astype(f32) + rbuf[2, rows, :].astype(f32))
        )

    for rdma in sends:
        rdma.wait_send()


def kernel(x, Wq, K_ext, V_ext, Wo):
    x16 = x.astype(jnp.bfloat16)
    wq16 = Wq.astype(jnp.bfloat16)
    wo16 = Wo.astype(jnp.bfloat16)
    k16 = jnp.transpose(K_ext[0], (1, 0, 2)).astype(jnp.bfloat16)
    v16 = jnp.transpose(V_ext[0], (1, 0, 2)).astype(jnp.bfloat16)

    chunk_shape = jax.ShapeDtypeStruct((H_PER, N_DEV, SKV_PER, DH),
                                       jnp.bfloat16)
    part, _, _ = pl.pallas_call(
        _attn_body,
        out_shape=[
            jax.ShapeDtypeStruct((SQ, DM), jnp.float32),
            chunk_shape,
            chunk_shape,
        ],
        in_specs=[
            pl.BlockSpec(memory_space=pltpu.VMEM),
            pl.BlockSpec(memory_space=pltpu.VMEM),
            pl.BlockSpec(memory_space=pltpu.VMEM),
            pl.BlockSpec(memory_space=pl.ANY),
            pl.BlockSpec(memory_space=pl.ANY),
        ],
        out_specs=[
            pl.BlockSpec(memory_space=pltpu.VMEM),
            pl.BlockSpec(memory_space=pl.ANY),
            pl.BlockSpec(memory_space=pl.ANY),
        ],
        scratch_shapes=[
            pltpu.VMEM((2, KBLK, DH), jnp.bfloat16),
            pltpu.VMEM((2, KBLK, DH), jnp.bfloat16),
            pltpu.SemaphoreType.DMA((N_DEV - 1, 2)),
            pltpu.SemaphoreType.DMA((N_DEV, 2)),
            pltpu.SemaphoreType.DMA((2,)),
            pltpu.SemaphoreType.DMA((2, 2)),
        ],
        compiler_params=pltpu.CompilerParams(collective_id=0),
    )(x16, wq16, wo16, k16, v16)

    out = pl.pallas_call(
        _allreduce_body,
        out_shape=jax.ShapeDtypeStruct((1, SQ, DM), jnp.float32),
        in_specs=[pl.BlockSpec(memory_space=pltpu.VMEM)],
        out_specs=pl.BlockSpec(memory_space=pltpu.VMEM),
        scratch_shapes=[
            pltpu.VMEM((N_DEV - 1, SQ, DM), jnp.bfloat16),
            pltpu.SemaphoreType.DMA((N_DEV - 1,)),
            pltpu.SemaphoreType.DMA((N_DEV - 1,)),
        ],
        compiler_params=pltpu.CompilerParams(collective_id=1),
    )(part.astype(jnp.bfloat16))
    return out


# device time: 615705 ns/iter; 1.7161x vs baseline; 1.0081x over previous
import jax
import jax.numpy as jnp
from jax import lax
from jax.experimental import pallas as pl
from jax.experimental.pallas import tpu as pltpu

N_DEV = 4
SQ = 2048
SKV_PER = 2048
SKV = N_DEV * SKV_PER
H_PER = 8
DH = 128
DM = 1024
KBLK = 512
QBLK = 512
N_KB = SKV // KBLK
KB_PER_CHUNK = SKV_PER // KBLK
N_KB_A = (SQ + 512) // KBLK
G = 32
SCALE = 0.08838834764831843


def _attn_body(x_ref, wq_ref, wo_ref, k_hbm, v_hbm, out_ref, kc, vc,
               kbuf, vbuf, send_sems, recv_sems, copy_sems, stream_sems):
    me = lax.axis_index("i")

    bsem = pltpu.get_barrier_semaphore()
    for p in range(N_DEV - 1):
        j = (me + 1 + p) % N_DEV
        pl.semaphore_signal(bsem, 1, device_id=(j,),
                            device_id_type=pl.DeviceIdType.MESH)
    pl.semaphore_wait(bsem, N_DEV - 1)

    sends = []
    for sub in range(KB_PER_CHUNK):
        rows = pl.ds(sub * KBLK, KBLK)
        for t, (src, chunks) in enumerate(((k_hbm, kc), (v_hbm, vc))):
            for p in range(N_DEV - 1):
                j = (me + 1 + p) % N_DEV
                rdma = pltpu.make_async_remote_copy(
                    src_ref=src.at[pl.ds(H_PER * j, H_PER), rows, :],
                    dst_ref=chunks.at[:, me, rows, :],
                    send_sem=send_sems.at[p, t, sub],
                    recv_sem=recv_sems.at[me, t, sub],
                    device_id=(j,),
                    device_id_type=pl.DeviceIdType.MESH,
                )
                rdma.start()
                sends.append(rdma)

    for sub in range(KB_PER_CHUNK):
        rows = pl.ds(sub * KBLK, KBLK)
        for t, (src, chunks) in enumerate(((k_hbm, kc), (v_hbm, vc))):
            pltpu.make_async_copy(
                src.at[pl.ds(H_PER * me, H_PER), rows, :],
                chunks.at[:, me, rows, :],
                copy_sems.at[t, sub]).start()

    q16 = (jnp.dot(x_ref[0], wq_ref[...],
                   preferred_element_type=jnp.float32)
           * SCALE).astype(jnp.bfloat16)

    def wait_sub(kb):
        c = kb // KB_PER_CHUNK
        sub = kb % KB_PER_CHUNK
        rows = pl.ds(0, KBLK)
        for t, (src, chunks) in enumerate(((k_hbm, kc), (v_hbm, vc))):
            @pl.when(me == c)
            def _(t=t, src=src, chunks=chunks):
                pltpu.make_async_copy(
                    src.at[pl.ds(0, H_PER), rows, :],
                    chunks.at[:, c, rows, :],
                    copy_sems.at[t, sub]).wait()

            @pl.when(me != c)
            def _(t=t, src=src, chunks=chunks):
                recv = pltpu.make_async_remote_copy(
                    src_ref=src.at[pl.ds(0, H_PER), rows, :],
                    dst_ref=chunks.at[:, c, rows, :],
                    send_sem=send_sems.at[0, t, 0],
                    recv_sem=recv_sems.at[c, t, sub],
                    device_id=(me,),
                    device_id_type=pl.DeviceIdType.MESH,
                )
                recv.wait_recv()

    def stream_desc(h, kb, b):
        chunk = kb // KB_PER_CHUNK
        row = (kb % KB_PER_CHUNK) * KBLK
        return [
            pltpu.make_async_copy(
                chunks.at[h, chunk, pl.ds(row, KBLK), :],
                buf.at[b], stream_sems.at[b, t])
            for t, (chunks, buf) in enumerate(((kc, kbuf), (vc, vbuf)))
        ]

    wait_sub(0)
    for cp in stream_desc(0, 0, 0):
        cp.start()

    strips = []
    for h in range(H_PER):
        q_h = q16[:, h * DH:(h + 1) * DH]

        def kb_a_body(kb, carry, h=h, q_h=q_h):
            m, l, acc = carry
            b = lax.rem(h * N_KB_A + kb, 2)
            for cp in stream_desc(h, kb, b):
                cp.wait()

            @pl.when(kb + 1 < N_KB_A)
            def _():
                if h == 0:
                    wait_sub(kb + 1)
                for cp in stream_desc(h, kb + 1, 1 - b):
                    cp.start()

            kblk = kbuf[b]
            vblk = vbuf[b]
            s = jnp.dot(q_h, kblk.T, preferred_element_type=jnp.float32)
            qi = lax.broadcasted_iota(jnp.int32, (SQ, KBLK), 0)
            ki = kb * KBLK + lax.broadcasted_iota(jnp.int32, (SQ, KBLK), 1)
            mask = (jnp.abs(qi - ki) <= 128) | (ki < G) | (qi < G)
            s = jnp.where(mask, s, -1e9)
            m_new = jnp.maximum(m, s.max(axis=1, keepdims=True))
            pw = jnp.exp(s - m_new)
            corr = jnp.exp(m - m_new)
            l_new = l * corr + pw.sum(axis=1, keepdims=True)
            acc_new = acc * corr + jnp.dot(
                pw.astype(jnp.bfloat16), vblk,
                preferred_element_type=jnp.float32)
            return m_new, l_new, acc_new

        m0 = jnp.full((SQ, 1), -1e30, jnp.float32)
        l0 = jnp.zeros((SQ, 1), jnp.float32)
        a0 = jnp.zeros((SQ, DH), jnp.float32)
        m, l, acc = lax.fori_loop(0, N_KB_A, kb_a_body, (m0, l0, a0))

        if h + 1 < H_PER:
            for cp in stream_desc(h + 1, 0, ((h + 1) * N_KB_A) % 2):
                cp.start()

        strips.append((m[0:G, :], l[0:G, :], acc[0:G, :]))
        ctx_h = (acc / l).astype(jnp.bfloat16)
        for rb in range(SQ // QBLK):
            rows = pl.ds(rb * QBLK, QBLK)
            contrib = jnp.dot(ctx_h[rb * QBLK:(rb + 1) * QBLK, :],
                              wo_ref[h * DH:(h + 1) * DH, :],
                              preferred_element_type=jnp.float32)
            if h == 0:
                out_ref[rows, :] = contrib
            else:
                out_ref[rows, :] = out_ref[rows, :] + contrib

    wait_sub(N_KB_A)
    for cp in stream_desc(0, N_KB_A, 0):
        cp.start()

    n_b = N_KB - N_KB_A
    fix = jnp.zeros((G, DM), jnp.float32)
    for h in range(H_PER):
        q_g = q16[0:G, h * DH:(h + 1) * DH]

        def kb_b_body(kb, carry, h=h, q_g=q_g):
            m, l, acc = carry
            b = lax.rem(h * n_b + kb - N_KB_A, 2)
            for cp in stream_desc(h, kb, b):
                cp.wait()

            @pl.when(kb + 1 < N_KB)
            def _():
                if h == 0:
                    wait_sub(kb + 1)
                for cp in stream_desc(h, kb + 1, 1 - b):
                    cp.start()

            kblk = kbuf[b]
            vblk = vbuf[b]
            s = jnp.dot(q_g, kblk.T, preferred_element_type=jnp.float32)
            m_new = jnp.maximum(m, s.max(axis=1, keepdims=True))
            pw = jnp.exp(s - m_new)
            corr = jnp.exp(m - m_new)
            l_new = l * corr + pw.sum(axis=1, keepdims=True)
            acc_new = acc * corr + jnp.dot(
                pw.astype(jnp.bfloat16), vblk,
                preferred_element_type=jnp.float32)
            return m_new, l_new, acc_new

        mg0 = jnp.full((G, 1), -1e30, jnp.float32)
        lg0 = jnp.zeros((G, 1), jnp.float32)
        ag0 = jnp.zeros((G, DH), jnp.float32)
        mg, lg, ag = lax.fori_loop(N_KB_A, N_KB, kb_b_body, (mg0, lg0, ag0))

        if h + 1 < H_PER:
            for cp in stream_desc(h + 1, N_KB_A, ((h + 1) * n_b) % 2):
                cp.start()

        m_a, l_a, a_a = strips[h]
        mc = jnp.maximum(m_a, mg)
        wa = jnp.exp(m_a - mc)
        wb = jnp.exp(mg - mc)
        ctx_g = ((a_a * wa + ag * wb) / (l_a * wa + lg * wb)
                 ).astype(jnp.bfloat16)
        fix = fix + jnp.dot(ctx_g, wo_ref[h * DH:(h + 1) * DH, :],
                            preferred_element_type=jnp.float32)

    out_ref[0:G, :] = fix

    for rdma in sends:
        rdma.wait_send()


def _allreduce_body(part_ref, out_ref, rbuf, ssems, rsems):
    me = lax.axis_index("i")

    bsem = pltpu.get_barrier_semaphore()
    for p in range(N_DEV - 1):
        j = (me + 1 + p) % N_DEV
        pl.semaphore_signal(bsem, 1, device_id=(j,),
                            device_id_type=pl.DeviceIdType.MESH)
    pl.semaphore_wait(bsem, N_DEV - 1)

    sends = []
    for p in range(N_DEV - 1):
        j = (me + 1 + p) % N_DEV
        slot_on_j = (me - j - 1) % N_DEV
        rdma = pltpu.make_async_remote_copy(
            src_ref=part_ref,
            dst_ref=rbuf.at[slot_on_j],
            send_sem=ssems.at[p],
            recv_sem=rsems.at[slot_on_j],
            device_id=(j,),
            device_id_type=pl.DeviceIdType.MESH,
        )
        rdma.start()
        sends.append(rdma)

    for d in range(N_DEV - 1):
        recv = pltpu.make_async_remote_copy(
            src_ref=part_ref,
            dst_ref=rbuf.at[d],
            send_sem=ssems.at[0],
            recv_sem=rsems.at[d],
            device_id=(me,),
            device_id_type=pl.DeviceIdType.MESH,
        )
        recv.wait_recv()

    f32 = jnp.float32
    for rb in range(SQ // QBLK):
        rows = pl.ds(rb * QBLK, QBLK)
        out_ref[0, rows, :] = (
            (part_ref[rows, :].astype(f32) + rbuf[0, rows, :].astype(f32))
            + (rbuf[1, rows, :].astype(f32) + rbuf[2, rows, :].astype(f32))
        )

    for rdma in sends:
        rdma.wait_send()


def kernel(x, Wq, K_ext, V_ext, Wo):
    x16 = x.astype(jnp.bfloat16)
    wq16 = Wq.astype(jnp.bfloat16)
    wo16 = Wo.astype(jnp.bfloat16)
    k16 = jnp.transpose(K_ext[0], (1, 0, 2)).astype(jnp.bfloat16)
    v16 = jnp.transpose(V_ext[0], (1, 0, 2)).astype(jnp.bfloat16)

    chunk_shape = jax.ShapeDtypeStruct((H_PER, N_DEV, SKV_PER, DH),
                                       jnp.bfloat16)
    part, _, _ = pl.pallas_call(
        _attn_body,
        out_shape=[
            jax.ShapeDtypeStruct((SQ, DM), jnp.float32),
            chunk_shape,
            chunk_shape,
        ],
        in_specs=[
            pl.BlockSpec(memory_space=pltpu.VMEM),
            pl.BlockSpec(memory_space=pltpu.VMEM),
            pl.BlockSpec(memory_space=pltpu.VMEM),
            pl.BlockSpec(memory_space=pl.ANY),
            pl.BlockSpec(memory_space=pl.ANY),
        ],
        out_specs=[
            pl.BlockSpec(memory_space=pltpu.VMEM),
            pl.BlockSpec(memory_space=pl.ANY),
            pl.BlockSpec(memory_space=pl.ANY),
        ],
        scratch_shapes=[
            pltpu.VMEM((2, KBLK, DH), jnp.bfloat16),
            pltpu.VMEM((2, KBLK, DH), jnp.bfloat16),
            pltpu.SemaphoreType.DMA((N_DEV - 1, 2, KB_PER_CHUNK)),
            pltpu.SemaphoreType.DMA((N_DEV, 2, KB_PER_CHUNK)),
            pltpu.SemaphoreType.DMA((2, KB_PER_CHUNK)),
            pltpu.SemaphoreType.DMA((2, 2)),
        ],
        compiler_params=pltpu.CompilerParams(collective_id=0),
    )(x16, wq16, wo16, k16, v16)

    out = pl.pallas_call(
        _allreduce_body,
        out_shape=jax.ShapeDtypeStruct((1, SQ, DM), jnp.float32),
        in_specs=[pl.BlockSpec(memory_space=pltpu.VMEM)],
        out_specs=pl.BlockSpec(memory_space=pltpu.VMEM),
        scratch_shapes=[
            pltpu.VMEM((N_DEV - 1, SQ, DM), jnp.bfloat16),
            pltpu.SemaphoreType.DMA((N_DEV - 1,)),
            pltpu.SemaphoreType.DMA((N_DEV - 1,)),
        ],
        compiler_params=pltpu.CompilerParams(collective_id=1),
    )(part.astype(jnp.bfloat16))
    return out


# device time: 549090 ns/iter; 1.9243x vs baseline; 1.1213x over previous
import jax
import jax.numpy as jnp
from jax import lax
from jax.experimental import pallas as pl
from jax.experimental.pallas import tpu as pltpu

N_DEV = 4
SQ = 2048
SKV_PER = 2048
SKV = N_DEV * SKV_PER
H_PER = 8
DH = 128
DM = 1024
KBLK = 512
QBLK = 512
N_KB = SKV // KBLK
KB_PER_CHUNK = SKV_PER // KBLK
N_KB_A = (SQ + 512) // KBLK
G = 32
SCALE = 0.08838834764831843


def _attn_body(x_ref, wq_ref, wo_ref, k_hbm, v_hbm, out_ref, kc, vc,
               kbuf, vbuf, kb8, vb8, send_sems, recv_sems, copy_sems,
               stream_sems, stream8_sems):
    me = lax.axis_index("i")

    bsem = pltpu.get_barrier_semaphore()
    for p in range(N_DEV - 1):
        j = (me + 1 + p) % N_DEV
        pl.semaphore_signal(bsem, 1, device_id=(j,),
                            device_id_type=pl.DeviceIdType.MESH)
    pl.semaphore_wait(bsem, N_DEV - 1)

    sends = []
    for sub in range(KB_PER_CHUNK):
        rows = pl.ds(sub * KBLK, KBLK)
        for t, (src, chunks) in enumerate(((k_hbm, kc), (v_hbm, vc))):
            for p in range(N_DEV - 1):
                j = (me + 1 + p) % N_DEV
                rdma = pltpu.make_async_remote_copy(
                    src_ref=src.at[pl.ds(H_PER * j, H_PER), rows, :],
                    dst_ref=chunks.at[:, me, rows, :],
                    send_sem=send_sems.at[p, t, sub],
                    recv_sem=recv_sems.at[me, t, sub],
                    device_id=(j,),
                    device_id_type=pl.DeviceIdType.MESH,
                )
                rdma.start()
                sends.append(rdma)

    for sub in range(KB_PER_CHUNK):
        rows = pl.ds(sub * KBLK, KBLK)
        for t, (src, chunks) in enumerate(((k_hbm, kc), (v_hbm, vc))):
            pltpu.make_async_copy(
                src.at[pl.ds(H_PER * me, H_PER), rows, :],
                chunks.at[:, me, rows, :],
                copy_sems.at[t, sub]).start()

    q16 = (jnp.dot(x_ref[0], wq_ref[...],
                   preferred_element_type=jnp.float32)
           * SCALE).astype(jnp.bfloat16)

    def wait_sub(kb):
        c = kb // KB_PER_CHUNK
        sub = kb % KB_PER_CHUNK
        rows = pl.ds(0, KBLK)
        for t, (src, chunks) in enumerate(((k_hbm, kc), (v_hbm, vc))):
            @pl.when(me == c)
            def _(t=t, src=src, chunks=chunks):
                pltpu.make_async_copy(
                    src.at[pl.ds(0, H_PER), rows, :],
                    chunks.at[:, c, rows, :],
                    copy_sems.at[t, sub]).wait()

            @pl.when(me != c)
            def _(t=t, src=src, chunks=chunks):
                recv = pltpu.make_async_remote_copy(
                    src_ref=src.at[pl.ds(0, H_PER), rows, :],
                    dst_ref=chunks.at[:, c, rows, :],
                    send_sem=send_sems.at[0, t, 0],
                    recv_sem=recv_sems.at[c, t, sub],
                    device_id=(me,),
                    device_id_type=pl.DeviceIdType.MESH,
                )
                recv.wait_recv()

    def stream_desc(h, kb, b):
        chunk = kb // KB_PER_CHUNK
        row = (kb % KB_PER_CHUNK) * KBLK
        return [
            pltpu.make_async_copy(
                chunks.at[h, chunk, pl.ds(row, KBLK), :],
                buf.at[b], stream_sems.at[b, t])
            for t, (chunks, buf) in enumerate(((kc, kbuf), (vc, vbuf)))
        ]

    wait_sub(0)
    for cp in stream_desc(0, 0, 0):
        cp.start()

    strips = []
    for h in range(H_PER):
        q_h = q16[:, h * DH:(h + 1) * DH]

        def kb_a_body(kb, carry, h=h, q_h=q_h):
            m, l, acc = carry
            b = lax.rem(h * N_KB_A + kb, 2)
            for cp in stream_desc(h, kb, b):
                cp.wait()

            @pl.when(kb + 1 < N_KB_A)
            def _():
                if h == 0:
                    wait_sub(kb + 1)
                for cp in stream_desc(h, kb + 1, 1 - b):
                    cp.start()

            kblk = kbuf[b]
            vblk = vbuf[b]
            s = jnp.dot(q_h, kblk.T, preferred_element_type=jnp.float32)
            qi = lax.broadcasted_iota(jnp.int32, (SQ, KBLK), 0)
            ki = kb * KBLK + lax.broadcasted_iota(jnp.int32, (SQ, KBLK), 1)
            mask = (jnp.abs(qi - ki) <= 128) | (ki < G) | (qi < G)
            s = jnp.where(mask, s, -1e9)
            m_new = jnp.maximum(m, s.max(axis=1, keepdims=True))
            pw = jnp.exp(s - m_new)
            corr = jnp.exp(m - m_new)
            l_new = l * corr + pw.sum(axis=1, keepdims=True)
            acc_new = acc * corr + jnp.dot(
                pw.astype(jnp.bfloat16), vblk,
                preferred_element_type=jnp.float32)
            return m_new, l_new, acc_new

        m0 = jnp.full((SQ, 1), -1e30, jnp.float32)
        l0 = jnp.zeros((SQ, 1), jnp.float32)
        a0 = jnp.zeros((SQ, DH), jnp.float32)
        m, l, acc = lax.fori_loop(0, N_KB_A, kb_a_body, (m0, l0, a0))

        if h + 1 < H_PER:
            for cp in stream_desc(h + 1, 0, ((h + 1) * N_KB_A) % 2):
                cp.start()

        strips.append((m[0:G, :], l[0:G, :], acc[0:G, :]))
        ctx_h = (acc / l).astype(jnp.bfloat16)
        for rb in range(SQ // QBLK):
            rows = pl.ds(rb * QBLK, QBLK)
            contrib = jnp.dot(ctx_h[rb * QBLK:(rb + 1) * QBLK, :],
                              wo_ref[h * DH:(h + 1) * DH, :],
                              preferred_element_type=jnp.float32)
            if h == 0:
                out_ref[rows, :] = contrib
            else:
                out_ref[rows, :] = out_ref[rows, :] + contrib

    def stream8_desc(kb, b):
        chunk = kb // KB_PER_CHUNK
        row = (kb % KB_PER_CHUNK) * KBLK
        return [
            pltpu.make_async_copy(
                chunks.at[:, chunk, pl.ds(row, KBLK), :],
                buf.at[b], stream8_sems.at[b, t])
            for t, (chunks, buf) in enumerate(((kc, kb8), (vc, vb8)))
        ]

    wait_sub(N_KB_A)
    for cp in stream8_desc(N_KB_A, 0):
        cp.start()

    def kb_b_body(kb, states):
        b = lax.rem(kb - N_KB_A, 2)
        for cp in stream8_desc(kb, b):
            cp.wait()

        @pl.when(kb + 1 < N_KB)
        def _():
            wait_sub(kb + 1)
            for cp in stream8_desc(kb + 1, 1 - b):
                cp.start()

        new_states = []
        for h in range(H_PER):
            m, l, acc = states[h]
            q_g = q16[0:G, h * DH:(h + 1) * DH]
            s = jnp.dot(q_g, kb8[b, h].T, preferred_element_type=jnp.float32)
            m_new = jnp.maximum(m, s.max(axis=1, keepdims=True))
            pw = jnp.exp(s - m_new)
            corr = jnp.exp(m - m_new)
            l_new = l * corr + pw.sum(axis=1, keepdims=True)
            acc_new = acc * corr + jnp.dot(
                pw.astype(jnp.bfloat16), vb8[b, h],
                preferred_element_type=jnp.float32)
            new_states.append((m_new, l_new, acc_new))
        return tuple(new_states)

    states0 = tuple(
        (jnp.full((G, 1), -1e30, jnp.float32),
         jnp.zeros((G, 1), jnp.float32),
         jnp.zeros((G, DH), jnp.float32))
        for _ in range(H_PER))
    states = lax.fori_loop(N_KB_A, N_KB, kb_b_body, states0)

    fix = jnp.zeros((G, DM), jnp.float32)
    for h in range(H_PER):
        mg, lg, ag = states[h]
        m_a, l_a, a_a = strips[h]
        mc = jnp.maximum(m_a, mg)
        wa = jnp.exp(m_a - mc)
        wb = jnp.exp(mg - mc)
        ctx_g = ((a_a * wa + ag * wb) / (l_a * wa + lg * wb)
                 ).astype(jnp.bfloat16)
        fix = fix + jnp.dot(ctx_g, wo_ref[h * DH:(h + 1) * DH, :],
                            preferred_element_type=jnp.float32)

    out_ref[0:G, :] = fix

    for rdma in sends:
        rdma.wait_send()


def _allreduce_body(part_ref, out_ref, rbuf, ssems, rsems):
    me = lax.axis_index("i")

    bsem = pltpu.get_barrier_semaphore()
    for p in range(N_DEV - 1):
        j = (me + 1 + p) % N_DEV
        pl.semaphore_signal(bsem, 1, device_id=(j,),
                            device_id_type=pl.DeviceIdType.MESH)
    pl.semaphore_wait(bsem, N_DEV - 1)

    sends = []
    for p in range(N_DEV - 1):
        j = (me + 1 + p) % N_DEV
        slot_on_j = (me - j - 1) % N_DEV
        rdma = pltpu.make_async_remote_copy(
            src_ref=part_ref,
            dst_ref=rbuf.at[slot_on_j],
            send_sem=ssems.at[p],
            recv_sem=rsems.at[slot_on_j],
            device_id=(j,),
            device_id_type=pl.DeviceIdType.MESH,
        )
        rdma.start()
        sends.append(rdma)

    for d in range(N_DEV - 1):
        recv = pltpu.make_async_remote_copy(
            src_ref=part_ref,
            dst_ref=rbuf.at[d],
            send_sem=ssems.at[0],
            recv_sem=rsems.at[d],
            device_id=(me,),
            device_id_type=pl.DeviceIdType.MESH,
        )
        recv.wait_recv()

    f32 = jnp.float32
    for rb in range(SQ // QBLK):
        rows = pl.ds(rb * QBLK, QBLK)
        out_ref[0, rows, :] = (
            (part_ref[rows, :].astype(f32) + rbuf[0, rows, :].astype(f32))
            + (rbuf[1, rows, :].astype(f32) + rbuf[2, rows, :].astype(f32))
        )

    for rdma in sends:
        rdma.wait_send()


def kernel(x, Wq, K_ext, V_ext, Wo):
    x16 = x.astype(jnp.bfloat16)
    wq16 = Wq.astype(jnp.bfloat16)
    wo16 = Wo.astype(jnp.bfloat16)
    k16 = jnp.transpose(K_ext[0], (1, 0, 2)).astype(jnp.bfloat16)
    v16 = jnp.transpose(V_ext[0], (1, 0, 2)).astype(jnp.bfloat16)

    chunk_shape = jax.ShapeDtypeStruct((H_PER, N_DEV, SKV_PER, DH),
                                       jnp.bfloat16)
    part, _, _ = pl.pallas_call(
        _attn_body,
        out_shape=[
            jax.ShapeDtypeStruct((SQ, DM), jnp.float32),
            chunk_shape,
            chunk_shape,
        ],
        in_specs=[
            pl.BlockSpec(memory_space=pltpu.VMEM),
            pl.BlockSpec(memory_space=pltpu.VMEM),
            pl.BlockSpec(memory_space=pltpu.VMEM),
            pl.BlockSpec(memory_space=pl.ANY),
            pl.BlockSpec(memory_space=pl.ANY),
        ],
        out_specs=[
            pl.BlockSpec(memory_space=pltpu.VMEM),
            pl.BlockSpec(memory_space=pl.ANY),
            pl.BlockSpec(memory_space=pl.ANY),
        ],
        scratch_shapes=[
            pltpu.VMEM((2, KBLK, DH), jnp.bfloat16),
            pltpu.VMEM((2, KBLK, DH), jnp.bfloat16),
            pltpu.VMEM((2, H_PER, KBLK, DH), jnp.bfloat16),
            pltpu.VMEM((2, H_PER, KBLK, DH), jnp.bfloat16),
            pltpu.SemaphoreType.DMA((N_DEV - 1, 2, KB_PER_CHUNK)),
            pltpu.SemaphoreType.DMA((N_DEV, 2, KB_PER_CHUNK)),
            pltpu.SemaphoreType.DMA((2, KB_PER_CHUNK)),
            pltpu.SemaphoreType.DMA((2, 2)),
            pltpu.SemaphoreType.DMA((2, 2)),
        ],
        compiler_params=pltpu.CompilerParams(collective_id=0),
    )(x16, wq16, wo16, k16, v16)

    out = pl.pallas_call(
        _allreduce_body,
        out_shape=jax.ShapeDtypeStruct((1, SQ, DM), jnp.float32),
        in_specs=[pl.BlockSpec(memory_space=pltpu.VMEM)],
        out_specs=pl.BlockSpec(memory_space=pltpu.VMEM),
        scratch_shapes=[
            pltpu.VMEM((N_DEV - 1, SQ, DM), jnp.bfloat16),
            pltpu.SemaphoreType.DMA((N_DEV - 1,)),
            pltpu.SemaphoreType.DMA((N_DEV - 1,)),
        ],
        compiler_params=pltpu.CompilerParams(collective_id=1),
    )(part.astype(jnp.bfloat16))
    return out


# device time: 547924 ns/iter; 1.9284x vs baseline; 1.0021x over previous
import jax
import jax.numpy as jnp
from jax import lax
from jax.experimental import pallas as pl
from jax.experimental.pallas import tpu as pltpu

N_DEV = 4
SQ = 2048
SKV_PER = 2048
SKV = N_DEV * SKV_PER
H_PER = 8
DH = 128
DM = 1024
KBLK = 512
QBLK = 512
N_KB = SKV // KBLK
KB_PER_CHUNK = SKV_PER // KBLK
N_KB_A = (SQ + 512) // KBLK
G = 32
SCALE = 0.08838834764831843


def _attn_body(x_ref, wq_ref, wo_ref, k_hbm, v_hbm, out_ref, kc, vc,
               kA, vA, kb8, vb8, send_sems, recv_sems, copy_sems,
               prefill_sems, stream8_sems):
    me = lax.axis_index("i")

    bsem = pltpu.get_barrier_semaphore()
    for p in range(N_DEV - 1):
        j = (me + 1 + p) % N_DEV
        pl.semaphore_signal(bsem, 1, device_id=(j,),
                            device_id_type=pl.DeviceIdType.MESH)
    pl.semaphore_wait(bsem, N_DEV - 1)

    sends = []
    for sub in range(KB_PER_CHUNK):
        rows = pl.ds(sub * KBLK, KBLK)
        for t, (src, chunks) in enumerate(((k_hbm, kc), (v_hbm, vc))):
            for p in range(N_DEV - 1):
                j = (me + 1 + p) % N_DEV
                rdma = pltpu.make_async_remote_copy(
                    src_ref=src.at[pl.ds(H_PER * j, H_PER), rows, :],
                    dst_ref=chunks.at[:, me, rows, :],
                    send_sem=send_sems.at[p, t, sub],
                    recv_sem=recv_sems.at[me, t, sub],
                    device_id=(j,),
                    device_id_type=pl.DeviceIdType.MESH,
                )
                rdma.start()
                sends.append(rdma)

    for sub in range(KB_PER_CHUNK):
        rows = pl.ds(sub * KBLK, KBLK)
        for t, (src, chunks) in enumerate(((k_hbm, kc), (v_hbm, vc))):
            pltpu.make_async_copy(
                src.at[pl.ds(H_PER * me, H_PER), rows, :],
                chunks.at[:, me, rows, :],
                copy_sems.at[t, sub]).start()

    q16 = (jnp.dot(x_ref[0], wq_ref[...],
                   preferred_element_type=jnp.float32)
           * SCALE).astype(jnp.bfloat16)

    def wait_sub(kb):
        c = kb // KB_PER_CHUNK
        sub = kb % KB_PER_CHUNK
        rows = pl.ds(0, KBLK)
        for t, (src, chunks) in enumerate(((k_hbm, kc), (v_hbm, vc))):
            @pl.when(me == c)
            def _(t=t, src=src, chunks=chunks):
                pltpu.make_async_copy(
                    src.at[pl.ds(0, H_PER), rows, :],
                    chunks.at[:, c, rows, :],
                    copy_sems.at[t, sub]).wait()

            @pl.when(me != c)
            def _(t=t, src=src, chunks=chunks):
                recv = pltpu.make_async_remote_copy(
                    src_ref=src.at[pl.ds(0, H_PER), rows, :],
                    dst_ref=chunks.at[:, c, rows, :],
                    send_sem=send_sems.at[0, t, 0],
                    recv_sem=recv_sems.at[c, t, sub],
                    device_id=(me,),
                    device_id_type=pl.DeviceIdType.MESH,
                )
                recv.wait_recv()

    filled = set()

    def ensure_kb(kb):
        if kb in filled:
            return
        filled.add(kb)
        wait_sub(kb)
        chunk = kb // KB_PER_CHUNK
        row = (kb % KB_PER_CHUNK) * KBLK
        for t, (chunks, buf) in enumerate(((kc, kA), (vc, vA))):
            cp = pltpu.make_async_copy(
                chunks.at[:, chunk, pl.ds(row, KBLK), :],
                buf.at[:, pl.ds(kb * KBLK, KBLK), :],
                prefill_sems.at[t, kb])
            cp.start()
            cp.wait()

    KBN = (2, 3, 4, 5)
    N_QB = SQ // QBLK

    strips = []
    for h in range(H_PER):
        for qb in range(N_QB):
            if h == 0:
                for kb in range(KBN[qb]):
                    ensure_kb(kb)
            rows_lo = qb * QBLK
            q_qb = q16[rows_lo:rows_lo + QBLK, h * DH:(h + 1) * DH]

            def kb_a_body(kb, carry, h=h, q_qb=q_qb, rows_lo=rows_lo):
                m, l, acc = carry
                kblk = kA[h, pl.ds(kb * KBLK, KBLK), :]
                vblk = vA[h, pl.ds(kb * KBLK, KBLK), :]
                s = jnp.dot(q_qb, kblk.T, preferred_element_type=jnp.float32)
                qi = rows_lo + lax.broadcasted_iota(
                    jnp.int32, (QBLK, KBLK), 0)
                ki = kb * KBLK + lax.broadcasted_iota(
                    jnp.int32, (QBLK, KBLK), 1)
                mask = ((jnp.abs(qi - ki) <= 128) | (ki < G)
                        | ((qi < G) & (ki < 1024)))
                s = jnp.where(mask, s, -1e9)
                m_new = jnp.maximum(m, s.max(axis=1, keepdims=True))
                pw = jnp.exp(s - m_new)
                corr = jnp.exp(m - m_new)
                l_new = l * corr + pw.sum(axis=1, keepdims=True)
                acc_new = acc * corr + jnp.dot(
                    pw.astype(jnp.bfloat16), vblk,
                    preferred_element_type=jnp.float32)
                return m_new, l_new, acc_new

            m0 = jnp.full((QBLK, 1), -1e30, jnp.float32)
            l0 = jnp.zeros((QBLK, 1), jnp.float32)
            a0 = jnp.zeros((QBLK, DH), jnp.float32)
            m, l, acc = lax.fori_loop(0, KBN[qb], kb_a_body, (m0, l0, a0))

            if qb == 0:
                strips.append((m[0:G, :], l[0:G, :], acc[0:G, :]))
            ctx_qb = (acc / l).astype(jnp.bfloat16)
            rows = pl.ds(rows_lo, QBLK)
            contrib = jnp.dot(ctx_qb, wo_ref[h * DH:(h + 1) * DH, :],
                              preferred_element_type=jnp.float32)
            if h == 0:
                out_ref[rows, :] = contrib
            else:
                out_ref[rows, :] = out_ref[rows, :] + contrib

    PB0 = 1024 // KBLK

    def stream8_desc(kb, b):
        chunk = kb // KB_PER_CHUNK
        row = (kb % KB_PER_CHUNK) * KBLK
        return [
            pltpu.make_async_copy(
                chunks.at[:, chunk, pl.ds(row, KBLK), :],
                buf.at[b], stream8_sems.at[b, t])
            for t, (chunks, buf) in enumerate(((kc, kb8), (vc, vb8)))
        ]

    for cp in stream8_desc(PB0, 0):
        cp.start()

    def kb_b_body(kb, states):
        b = lax.rem(kb - PB0, 2)
        for cp in stream8_desc(kb, b):
            cp.wait()

        @pl.when((kb + 1 < N_KB) & (kb + 1 >= N_KB_A))
        def _():
            wait_sub(kb + 1)

        @pl.when(kb + 1 < N_KB)
        def _():
            for cp in stream8_desc(kb + 1, 1 - b):
                cp.start()

        new_states = []
        for h in range(H_PER):
            m, l, acc = states[h]
            q_g = q16[0:G, h * DH:(h + 1) * DH]
            s = jnp.dot(q_g, kb8[b, h].T, preferred_element_type=jnp.float32)
            m_new = jnp.maximum(m, s.max(axis=1, keepdims=True))
            pw = jnp.exp(s - m_new)
            corr = jnp.exp(m - m_new)
            l_new = l * corr + pw.sum(axis=1, keepdims=True)
            acc_new = acc * corr + jnp.dot(
                pw.astype(jnp.bfloat16), vb8[b, h],
                preferred_element_type=jnp.float32)
            new_states.append((m_new, l_new, acc_new))
        return tuple(new_states)

    states0 = tuple(
        (jnp.full((G, 1), -1e30, jnp.float32),
         jnp.zeros((G, 1), jnp.float32),
         jnp.zeros((G, DH), jnp.float32))
        for _ in range(H_PER))
    states = lax.fori_loop(PB0, N_KB, kb_b_body, states0)

    fix = jnp.zeros((G, DM), jnp.float32)
    for h in range(H_PER):
        mg, lg, ag = states[h]
        m_a, l_a, a_a = strips[h]
        mc = jnp.maximum(m_a, mg)
        wa = jnp.exp(m_a - mc)
        wb = jnp.exp(mg - mc)
        ctx_g = ((a_a * wa + ag * wb) / (l_a * wa + lg * wb)
                 ).astype(jnp.bfloat16)
        fix = fix + jnp.dot(ctx_g, wo_ref[h * DH:(h + 1) * DH, :],
                            preferred_element_type=jnp.float32)

    out_ref[0:G, :] = fix

    for rdma in sends:
        rdma.wait_send()


def _allreduce_body(part_ref, out_ref, rbuf, ssems, rsems):
    me = lax.axis_index("i")

    bsem = pltpu.get_barrier_semaphore()
    for p in range(N_DEV - 1):
        j = (me + 1 + p) % N_DEV
        pl.semaphore_signal(bsem, 1, device_id=(j,),
                            device_id_type=pl.DeviceIdType.MESH)
    pl.semaphore_wait(bsem, N_DEV - 1)

    sends = []
    for p in range(N_DEV - 1):
        j = (me + 1 + p) % N_DEV
        slot_on_j = (me - j - 1) % N_DEV
        rdma = pltpu.make_async_remote_copy(
            src_ref=part_ref,
            dst_ref=rbuf.at[slot_on_j],
            send_sem=ssems.at[p],
            recv_sem=rsems.at[slot_on_j],
            device_id=(j,),
            device_id_type=pl.DeviceIdType.MESH,
        )
        rdma.start()
        sends.append(rdma)

    for d in range(N_DEV - 1):
        recv = pltpu.make_async_remote_copy(
            src_ref=part_ref,
            dst_ref=rbuf.at[d],
            send_sem=ssems.at[0],
            recv_sem=rsems.at[d],
            device_id=(me,),
            device_id_type=pl.DeviceIdType.MESH,
        )
        recv.wait_recv()

    f32 = jnp.float32
    for rb in range(SQ // QBLK):
        rows = pl.ds(rb * QBLK, QBLK)
        out_ref[0, rows, :] = (
            (part_ref[rows, :].astype(f32) + rbuf[0, rows, :].astype(f32))
            + (rbuf[1, rows, :].astype(f32) + rbuf[2, rows, :].astype(f32))
        )

    for rdma in sends:
        rdma.wait_send()


def kernel(x, Wq, K_ext, V_ext, Wo):
    x16 = x.astype(jnp.bfloat16)
    wq16 = Wq.astype(jnp.bfloat16)
    wo16 = Wo.astype(jnp.bfloat16)
    k16 = jnp.transpose(K_ext[0], (1, 0, 2)).astype(jnp.bfloat16)
    v16 = jnp.transpose(V_ext[0], (1, 0, 2)).astype(jnp.bfloat16)

    chunk_shape = jax.ShapeDtypeStruct((H_PER, N_DEV, SKV_PER, DH),
                                       jnp.bfloat16)
    part, _, _ = pl.pallas_call(
        _attn_body,
        out_shape=[
            jax.ShapeDtypeStruct((SQ, DM), jnp.float32),
            chunk_shape,
            chunk_shape,
        ],
        in_specs=[
            pl.BlockSpec(memory_space=pltpu.VMEM),
            pl.BlockSpec(memory_space=pltpu.VMEM),
            pl.BlockSpec(memory_space=pltpu.VMEM),
            pl.BlockSpec(memory_space=pl.ANY),
            pl.BlockSpec(memory_space=pl.ANY),
        ],
        out_specs=[
            pl.BlockSpec(memory_space=pltpu.VMEM),
            pl.BlockSpec(memory_space=pl.ANY),
            pl.BlockSpec(memory_space=pl.ANY),
        ],
        scratch_shapes=[
            pltpu.VMEM((H_PER, N_KB_A * KBLK, DH), jnp.bfloat16),
            pltpu.VMEM((H_PER, N_KB_A * KBLK, DH), jnp.bfloat16),
            pltpu.VMEM((2, H_PER, KBLK, DH), jnp.bfloat16),
            pltpu.VMEM((2, H_PER, KBLK, DH), jnp.bfloat16),
            pltpu.SemaphoreType.DMA((N_DEV - 1, 2, KB_PER_CHUNK)),
            pltpu.SemaphoreType.DMA((N_DEV, 2, KB_PER_CHUNK)),
            pltpu.SemaphoreType.DMA((2, KB_PER_CHUNK)),
            pltpu.SemaphoreType.DMA((2, N_KB_A)),
            pltpu.SemaphoreType.DMA((2, 2)),
        ],
        compiler_params=pltpu.CompilerParams(collective_id=0),
    )(x16, wq16, wo16, k16, v16)

    out = pl.pallas_call(
        _allreduce_body,
        out_shape=jax.ShapeDtypeStruct((1, SQ, DM), jnp.float32),
        in_specs=[pl.BlockSpec(memory_space=pltpu.VMEM)],
        out_specs=pl.BlockSpec(memory_space=pltpu.VMEM),
        scratch_shapes=[
            pltpu.VMEM((N_DEV - 1, SQ, DM), jnp.bfloat16),
            pltpu.SemaphoreType.DMA((N_DEV - 1,)),
            pltpu.SemaphoreType.DMA((N_DEV - 1,)),
        ],
        compiler_params=pltpu.CompilerParams(collective_id=1),
    )(part.astype(jnp.bfloat16))
    return out
